# baseline (speedup 1.0000x reference)
import jax
import jax.numpy as jnp
from jax.experimental import pallas as pl
from jax.experimental.pallas import tpu as pltpu

DIM = 128
K = 65536
BATCH = 4096
BLK = 512
NBLK = K // BLK
WBLKS = BATCH // BLK


def _p_blk(ptr_ref):
    p = jnp.clip(ptr_ref[0], 0, K - BATCH)
    return jax.lax.div(p, BLK)


def _body(ptr_ref, q_ref, keys_ref, out_ref, ptr_out_ref):
    j = pl.program_id(0)
    pb = _p_blk(ptr_ref)
    in_win = (j >= pb) & (j < pb + WBLKS)

    @pl.when(in_win)
    def _():
        out_ref[...] = keys_ref[...].T

    @pl.when(jnp.logical_not(in_win))
    def _():
        out_ref[...] = q_ref[...]

    @pl.when(j == 0)
    def _():
        ptr_out_ref[0] = jax.lax.rem(ptr_ref[0] + BATCH, K)


def kernel(keys, queue, queue_ptr):
    ptr = queue_ptr.astype(jnp.int32)
    grid_spec = pltpu.PrefetchScalarGridSpec(
        num_scalar_prefetch=1,
        grid=(NBLK,),
        in_specs=[
            pl.BlockSpec((DIM, BLK), lambda j, pref: (0, j)),
            pl.BlockSpec(
                (BLK, DIM),
                lambda j, pref: (jnp.clip(j - _p_blk(pref), 0, WBLKS - 1), 0),
            ),
        ],
        out_specs=[
            pl.BlockSpec((DIM, BLK), lambda j, pref: (0, j)),
            pl.BlockSpec(memory_space=pltpu.SMEM),
        ],
    )
    new_queue, new_ptr = pl.pallas_call(
        _body,
        grid_spec=grid_spec,
        out_shape=[
            jax.ShapeDtypeStruct((DIM, K), jnp.float32),
            jax.ShapeDtypeStruct((1,), jnp.int32),
        ],
        compiler_params=pltpu.CompilerParams(
            dimension_semantics=("arbitrary",),
        ),
    )(ptr, queue, keys)
    return new_queue, new_ptr.astype(queue_ptr.dtype)
